# Initial kernel scaffold; baseline (speedup 1.0000x reference)
import functools

import jax
import jax.numpy as jnp
from jax import lax
from jax.experimental import pallas as pl
from jax.experimental.pallas import tpu as pltpu
from jax.experimental.pallas import tpu_sc as plsc

B = 4096
F = 26
V = 1000
D = 16
S = F * D
NC = 2
NS = 16
NW = NC * NS
BPW = B // NW
E = 4
IDXC = E * F
NCHUNK = BPW // E


def _build_table(weights, embeddings):
    wcol = jnp.zeros((1, V, D), weights.dtype)
    parts = []
    for f in range(F):
        wf = wcol.at[0, :, 0].set(weights[f, :, 0])
        rows = jnp.concatenate(
            [embeddings[f, :f], wf, embeddings[f, f:]], axis=0)
        parts.append(rows)
    t4 = jnp.stack(parts, axis=0)
    t = jnp.transpose(t4, (0, 2, 1, 3))
    return t.reshape(F * V, S)


@functools.partial(
    pl.kernel,
    out_type=jax.ShapeDtypeStruct((B,), jnp.float32),
    mesh=plsc.VectorSubcoreMesh(core_axis_name="c", subcore_axis_name="s"),
    scratch_types=[
        pltpu.VMEM((IDXC,), jnp.int32),
        pltpu.VMEM((IDXC,), jnp.int32),
        pltpu.VMEM((IDXC, S), jnp.float32),
        pltpu.VMEM((IDXC, S), jnp.float32),
        pltpu.VMEM((BPW,), jnp.float32),
        pltpu.SemaphoreType.DMA,
        pltpu.SemaphoreType.DMA,
    ],
)
def _ffm_sc(table_hbm, idx_hbm, out_hbm,
            idx0, idx1, rows0, rows1, outv, sem0, sem1):
    wid = lax.axis_index("s") * NC + lax.axis_index("c")
    base_ex = wid * BPW

    def start(chunk, idxbuf, rowbuf, sem):
        off = (base_ex + chunk * E) * F
        pltpu.sync_copy(idx_hbm.at[pl.ds(off, IDXC)], idxbuf)
        pltpu.async_copy(table_hbm.at[idxbuf], rowbuf, sem)

    def wait(idxbuf, rowbuf, sem):
        pltpu.make_async_copy(table_hbm.at[idxbuf], rowbuf, sem).wait()

    def compute(chunk, rowbuf):
        @pl.loop(0, E)
        def _(e):
            r0 = e * F
            acc = jnp.zeros((D,), jnp.float32)
            for i in range(F):
                acc = acc + rowbuf[r0 + i, pl.ds(D * i, D)]
            for i in range(F - 1):
                for j in range(i + 1, F):
                    a = rowbuf[r0 + i, pl.ds(D * j, D)]
                    b = rowbuf[r0 + j, pl.ds(D * i, D)]
                    acc = acc + a * b
            outv[chunk * E + e] = jnp.sum(acc, axis=0)

    start(0, idx0, rows0, sem0)

    @pl.loop(0, NCHUNK, step=2)
    def _(g):
        start(g + 1, idx1, rows1, sem1)
        wait(idx0, rows0, sem0)
        compute(g, rows0)

        @pl.when(g + 2 < NCHUNK)
        def _():
            start(g + 2, idx0, rows0, sem0)

        wait(idx1, rows1, sem1)
        compute(g + 1, rows1)

    pltpu.sync_copy(outv, out_hbm.at[pl.ds(base_ex, BPW)])


def kernel(indices, weights, embeddings, bias):
    table = _build_table(weights, embeddings)
    idx2 = (indices + jnp.arange(F, dtype=jnp.int32)[None, :] * V).reshape(-1)
    out = _ffm_sc(table, idx2)
    return out.reshape(B, 1) + bias


# speedup vs baseline: 21.9543x; 21.9543x over previous
import dataclasses
import functools

import jax
import jax.numpy as jnp
from jax import lax
from jax.experimental import pallas as pl
from jax.experimental.pallas import tpu as pltpu
from jax.experimental.pallas import tpu_sc as plsc

B = 4096
F = 26
V = 1000
D = 16
S = F * D
NC = 2
NS = 16
NW = NC * NS
BPW = B // NW
E = 4
IDXC = E * F
NCHUNK = BPW // E


def _build_table(weights, embeddings):
    wcol = jnp.zeros((1, V, D), weights.dtype)
    parts = []
    for f in range(F):
        wf = wcol.at[0, :, 0].set(weights[f, :, 0])
        rows = jnp.concatenate(
            [embeddings[f, :f], wf, embeddings[f, f:]], axis=0)
        parts.append(rows)
    t4 = jnp.stack(parts, axis=0)
    t = jnp.transpose(t4, (0, 2, 1, 3))
    return t.reshape(F * V, S)


@functools.cache
def _get_sc_kernel():
    mesh = plsc.VectorSubcoreMesh(core_axis_name="c", subcore_axis_name="s")
    cp = pltpu.CompilerParams()
    if "needs_layout_passes" in pltpu.CompilerParams.__dataclass_fields__:
        cp = dataclasses.replace(cp, needs_layout_passes=False)
    if "use_tc_tiling_on_sc" in pltpu.CompilerParams.__dataclass_fields__:
        cp = dataclasses.replace(cp, use_tc_tiling_on_sc=False)
    return functools.partial(
        pl.kernel,
        out_type=jax.ShapeDtypeStruct((B,), jnp.float32),
        mesh=mesh,
        compiler_params=cp,
        scratch_types=[
            pltpu.VMEM((IDXC,), jnp.int32),
            pltpu.VMEM((IDXC,), jnp.int32),
            pltpu.VMEM((IDXC, S), jnp.float32),
            pltpu.VMEM((IDXC, S), jnp.float32),
            pltpu.VMEM((BPW,), jnp.float32),
            pltpu.SMEM((BPW,), jnp.float32),
            pltpu.SemaphoreType.DMA,
            pltpu.SemaphoreType.DMA,
        ],
    )(_ffm_sc)


def _ffm_sc(table_hbm, idx_hbm, out_hbm,
            idx0, idx1, rows0, rows1, outv, outs, sem0, sem1):
    wid = lax.axis_index("s") * NC + lax.axis_index("c")
    base_ex = wid * BPW

    def start(chunk, idxbuf, rowbuf, sem):
        off = (base_ex + chunk * E) * F
        pltpu.sync_copy(idx_hbm.at[pl.ds(off, IDXC)], idxbuf)
        pltpu.async_copy(table_hbm.at[idxbuf], rowbuf, sem)

    def wait(idxbuf, rowbuf, sem):
        pltpu.make_async_copy(table_hbm.at[idxbuf], rowbuf, sem).wait()

    def compute(chunk, rowbuf):
        @pl.loop(0, E)
        def _(e):
            r0 = e * F
            acc = jnp.zeros((D,), jnp.float32)
            for i in range(F):
                acc = acc + rowbuf[r0 + i, pl.ds(D * i, D)]
            for i in range(F - 1):
                for j in range(i + 1, F):
                    a = rowbuf[r0 + i, pl.ds(D * j, D)]
                    b = rowbuf[r0 + j, pl.ds(D * i, D)]
                    acc = acc + a * b
            outs[chunk * E + e] = jnp.sum(acc, axis=0)

    start(0, idx0, rows0, sem0)

    @pl.loop(0, NCHUNK, step=2)
    def _(g):
        start(g + 1, idx1, rows1, sem1)
        wait(idx0, rows0, sem0)
        compute(g, rows0)

        @pl.when(g + 2 < NCHUNK)
        def _():
            start(g + 2, idx0, rows0, sem0)

        wait(idx1, rows1, sem1)
        compute(g + 1, rows1)

    lanes = jax.lax.iota(jnp.int32, 16)

    @pl.loop(0, BPW // 16)
    def _(k):
        v = jnp.zeros((16,), jnp.float32)
        for l in range(16):
            v = jnp.where(lanes == l, outs[k * 16 + l], v)
        outv[pl.ds(k * 16, 16)] = v

    pltpu.sync_copy(outv, out_hbm.at[pl.ds(base_ex, BPW)])


def kernel(indices, weights, embeddings, bias):
    table = _build_table(weights, embeddings)
    idx2 = (indices + jnp.arange(F, dtype=jnp.int32)[None, :] * V).reshape(-1)
    out = _get_sc_kernel()(table, idx2)
    return out.reshape(B, 1) + bias


# speedup vs baseline: 28.6331x; 1.3042x over previous
import dataclasses
import functools

import jax
import jax.numpy as jnp
from jax import lax
from jax.experimental import pallas as pl
from jax.experimental.pallas import tpu as pltpu
from jax.experimental.pallas import tpu_sc as plsc

B = 4096
F = 26
V = 1000
D = 16
NSLOT = F - 1
NC = 2
NS = 16
NW = NC * NS
BPW = B // NW
E = 4
IDXC = E * F
NCHUNK = BPW // E
NCH_G = B // E


@functools.cache
def _get_sc_kernel():
    mesh = plsc.VectorSubcoreMesh(core_axis_name="c", subcore_axis_name="s")
    cp = pltpu.CompilerParams()
    if "needs_layout_passes" in pltpu.CompilerParams.__dataclass_fields__:
        cp = dataclasses.replace(cp, needs_layout_passes=False)
    if "use_tc_tiling_on_sc" in pltpu.CompilerParams.__dataclass_fields__:
        cp = dataclasses.replace(cp, use_tc_tiling_on_sc=False)
    return functools.partial(
        pl.kernel,
        out_type=jax.ShapeDtypeStruct((B,), jnp.float32),
        mesh=mesh,
        compiler_params=cp,
        scratch_types=[
            pltpu.VMEM((NSLOT, IDXC), jnp.int32),
            pltpu.VMEM((NSLOT, IDXC), jnp.int32),
            pltpu.VMEM((IDXC,), jnp.int32),
            pltpu.VMEM((IDXC,), jnp.int32),
            pltpu.VMEM((NSLOT, IDXC, D), jnp.float32),
            pltpu.VMEM((NSLOT, IDXC, D), jnp.float32),
            pltpu.VMEM((IDXC, D), jnp.float32),
            pltpu.VMEM((IDXC, D), jnp.float32),
            pltpu.VMEM((BPW,), jnp.float32),
            pltpu.SMEM((BPW,), jnp.float32),
            pltpu.SemaphoreType.DMA,
            pltpu.SemaphoreType.DMA,
        ],
    )(_ffm_sc)


def _ffm_sc(etable, wtable, eidx_hbm, widx_hbm, out_hbm,
            eidx0, eidx1, widx0, widx1, ebuf0, ebuf1, wbuf0, wbuf1,
            outv, outs, sem0, sem1):
    wid = lax.axis_index("s") * NC + lax.axis_index("c")
    base_ex = wid * BPW
    base_ch = wid * NCHUNK

    def start(chunk, eidxb, widxb, ebuf, wbuf, sem):
        gc = base_ch + chunk
        pltpu.sync_copy(eidx_hbm.at[gc], eidxb)
        pltpu.sync_copy(widx_hbm.at[gc], widxb)
        for s in range(NSLOT):
            pltpu.async_copy(etable.at[eidxb.at[s]], ebuf.at[s], sem)
        pltpu.async_copy(wtable.at[widxb], wbuf, sem)

    def wait(eidxb, widxb, ebuf, wbuf, sem):
        for s in range(NSLOT):
            pltpu.make_async_copy(etable.at[eidxb.at[s]], ebuf.at[s], sem).wait()
        pltpu.make_async_copy(wtable.at[widxb], wbuf, sem).wait()

    def compute(chunk, ebuf, wbuf):
        @pl.loop(0, E)
        def _(e):
            r0 = e * F
            acc = jnp.zeros((D,), jnp.float32)
            for i in range(F):
                acc = acc + wbuf[r0 + i, :]
            for i in range(F - 1):
                for j in range(i + 1, F):
                    a = ebuf[j - 1, r0 + i, :]
                    b = ebuf[i, r0 + j, :]
                    acc = acc + a * b
            outs[chunk * E + e] = jnp.sum(acc, axis=0)

    start(0, eidx0, widx0, ebuf0, wbuf0, sem0)

    @pl.loop(0, NCHUNK, step=2)
    def _(g):
        start(g + 1, eidx1, widx1, ebuf1, wbuf1, sem1)
        wait(eidx0, widx0, ebuf0, wbuf0, sem0)
        compute(g, ebuf0, wbuf0)

        @pl.when(g + 2 < NCHUNK)
        def _():
            start(g + 2, eidx0, widx0, ebuf0, wbuf0, sem0)

        wait(eidx1, widx1, ebuf1, wbuf1, sem1)
        compute(g + 1, ebuf1, wbuf1)

    lanes = jax.lax.iota(jnp.int32, 16)

    @pl.loop(0, BPW // 16)
    def _(k):
        v = jnp.zeros((16,), jnp.float32)
        for l in range(16):
            v = jnp.where(lanes == l, outs[k * 16 + l], v)
        outv[pl.ds(k * 16, 16)] = v

    pltpu.sync_copy(outv, out_hbm.at[pl.ds(base_ex, BPW)])


def kernel(indices, weights, embeddings, bias):
    etable = embeddings.reshape(F * NSLOT * V, D)
    wtable = jnp.pad(weights, ((0, 0), (0, 0), (0, D - 1))).reshape(F * V, D)
    fbase = (jnp.arange(F, dtype=jnp.int32) * (NSLOT * V))[None, :]
    eidx = ((indices + fbase).reshape(NCH_G, 1, IDXC)
            + (jnp.arange(NSLOT, dtype=jnp.int32) * V)[None, :, None])
    widx = (indices + jnp.arange(F, dtype=jnp.int32)[None, :] * V).reshape(
        NCH_G, IDXC)
    out = _get_sc_kernel()(etable, wtable, eidx, widx)
    return out.reshape(B, 1) + bias


# speedup vs baseline: 30.0930x; 1.0510x over previous
import dataclasses
import functools

import jax
import jax.numpy as jnp
from jax import lax
from jax.experimental import pallas as pl
from jax.experimental.pallas import tpu as pltpu
from jax.experimental.pallas import tpu_sc as plsc

B = 4096
F = 26
V = 1000
D = 16
NSLOT = F - 1
NC = 2
NS = 16
NW = NC * NS
BPW = B // NW
E = 4
IDXC = E * F
NG = 7
IDXP = NG * 16
NCHUNK = BPW // E

_FPAT = [p % F for p in range(IDXP)]


@functools.cache
def _get_sc_kernel():
    mesh = plsc.VectorSubcoreMesh(core_axis_name="c", subcore_axis_name="s")
    cp = pltpu.CompilerParams()
    if "needs_layout_passes" in pltpu.CompilerParams.__dataclass_fields__:
        cp = dataclasses.replace(cp, needs_layout_passes=False)
    if "use_tc_tiling_on_sc" in pltpu.CompilerParams.__dataclass_fields__:
        cp = dataclasses.replace(cp, use_tc_tiling_on_sc=False)
    return functools.partial(
        pl.kernel,
        out_type=jax.ShapeDtypeStruct((B,), jnp.float32),
        mesh=mesh,
        compiler_params=cp,
        scratch_types=[
            pltpu.VMEM((IDXP,), jnp.int32),
            pltpu.VMEM((IDXP,), jnp.int32),
            pltpu.VMEM((NSLOT * IDXP,), jnp.int32),
            pltpu.VMEM((NSLOT * IDXP,), jnp.int32),
            pltpu.VMEM((IDXP,), jnp.int32),
            pltpu.VMEM((IDXP,), jnp.int32),
            pltpu.VMEM((NSLOT, IDXC, D), jnp.float32),
            pltpu.VMEM((NSLOT, IDXC, D), jnp.float32),
            pltpu.VMEM((IDXC, D), jnp.float32),
            pltpu.VMEM((IDXC, D), jnp.float32),
            pltpu.VMEM((BPW,), jnp.float32),
            pltpu.SMEM((BPW,), jnp.float32),
            pltpu.SemaphoreType.DMA,
            pltpu.SemaphoreType.DMA,
        ],
    )(_ffm_sc)


def _ffm_sc(etable, wtable, idx_hbm, out_hbm,
            iraw0, iraw1, eidx0, eidx1, widx0, widx1,
            ebuf0, ebuf1, wbuf0, wbuf1, outv, outs, sem0, sem1):
    wid = lax.axis_index("s") * NC + lax.axis_index("c")
    base_ex = wid * BPW

    lane16 = jax.lax.iota(jnp.int32, 16)
    fpat = [(lane16 + g * 16) % F for g in range(NG)]
    evec = [f * (NSLOT * V) for f in fpat]
    wvec = [f * V for f in fpat]

    zero16 = jnp.zeros((16,), jnp.int32)
    iraw0[pl.ds(IDXC - 8, 16)] = zero16
    iraw1[pl.ds(IDXC - 8, 16)] = zero16

    def start(chunk, iraw, eidx, widx, ebuf, wbuf, sem):
        off = (base_ex + chunk * E) * F
        pltpu.sync_copy(idx_hbm.at[pl.ds(off, IDXC)], iraw.at[pl.ds(0, IDXC)])
        base = [iraw[pl.ds(g * 16, 16)] for g in range(NG)]
        for g in range(NG):
            widx[pl.ds(g * 16, 16)] = base[g] + wvec[g]
        ebase = [base[g] + evec[g] for g in range(NG)]
        for s in range(NSLOT):
            for g in range(NG):
                eidx[pl.ds(s * IDXP + g * 16, 16)] = ebase[g] + s * V
        for s in range(NSLOT):
            pltpu.async_copy(
                etable.at[eidx.at[pl.ds(s * IDXP, IDXC)]], ebuf.at[s], sem)
        pltpu.async_copy(wtable.at[widx.at[pl.ds(0, IDXC)]], wbuf, sem)

    def wait(eidx, widx, ebuf, wbuf, sem):
        for s in range(NSLOT):
            pltpu.make_async_copy(
                etable.at[eidx.at[pl.ds(s * IDXP, IDXC)]], ebuf.at[s],
                sem).wait()
        pltpu.make_async_copy(wtable.at[widx.at[pl.ds(0, IDXC)]], wbuf,
                              sem).wait()

    def compute(chunk, ebuf, wbuf):
        @pl.loop(0, E)
        def _(e):
            r0 = e * F
            acc = jnp.zeros((D,), jnp.float32)
            for i in range(F):
                acc = acc + wbuf[r0 + i, :]
            for i in range(F - 1):
                for j in range(i + 1, F):
                    a = ebuf[j - 1, r0 + i, :]
                    b = ebuf[i, r0 + j, :]
                    acc = acc + a * b
            outs[chunk * E + e] = jnp.sum(acc, axis=0)

    start(0, iraw0, eidx0, widx0, ebuf0, wbuf0, sem0)

    @pl.loop(0, NCHUNK, step=2)
    def _(g):
        start(g + 1, iraw1, eidx1, widx1, ebuf1, wbuf1, sem1)
        wait(eidx0, widx0, ebuf0, wbuf0, sem0)
        compute(g, ebuf0, wbuf0)

        @pl.when(g + 2 < NCHUNK)
        def _():
            start(g + 2, iraw0, eidx0, widx0, ebuf0, wbuf0, sem0)

        wait(eidx1, widx1, ebuf1, wbuf1, sem1)
        compute(g + 1, ebuf1, wbuf1)

    lanes = jax.lax.iota(jnp.int32, 16)

    @pl.loop(0, BPW // 16)
    def _(k):
        v = jnp.zeros((16,), jnp.float32)
        for l in range(16):
            v = jnp.where(lanes == l, outs[k * 16 + l], v)
        outv[pl.ds(k * 16, 16)] = v

    pltpu.sync_copy(outv, out_hbm.at[pl.ds(base_ex, BPW)])


def kernel(indices, weights, embeddings, bias):
    etable = embeddings.reshape(F * NSLOT * V, D)
    wtable = jnp.pad(weights, ((0, 0), (0, 0), (0, D - 1))).reshape(F * V, D)
    out = _get_sc_kernel()(etable, wtable, indices.reshape(B * F))
    return out.reshape(B, 1) + bias
